# baseline (speedup 1.0000x reference)
import functools

import jax
import jax.numpy as jnp
from jax import lax
from jax.experimental import pallas as pl
from jax.experimental.pallas import tpu as pltpu
from jax.experimental.pallas import tpu_sc as plsc

_D = 32
_K = 8192
_N = 8192
_TN = 256
_GRID = _N // _TN
_COMMIT = 0.25

_NC = 2
_NS = 16
_NW = _NC * _NS
_BPW = _N // _NW

_DP = 128


def _dist_body(flat_ref, xsq_ref, w_ref, wsq_ref,
               dist_ref, idx_ref, loss_ref, acc_ref):
    i = pl.program_id(0)

    @pl.when(i == 0)
    def _init():
        acc_ref[...] = jnp.zeros_like(acc_ref)

    x = flat_ref[...]
    w = w_ref[...]
    dots = jax.lax.dot_general(x, w, (((1,), (1,)), ((), ())),
                               preferred_element_type=jnp.float32)
    dist = (xsq_ref[...] - 2.0 * dots) + wsq_ref[...]
    dist_ref[...] = dist

    dmin = jnp.min(dist, axis=1, keepdims=True)
    lanes = jax.lax.broadcasted_iota(jnp.int32, (_TN, _K), 1)
    idx = jnp.min(jnp.where(dist == dmin, lanes, _K), axis=1)
    idx_ref[...] = idx[:, None]
    acc_ref[...] += jnp.sum(dmin, keepdims=True).reshape(1, 1)

    @pl.when(i == _GRID - 1)
    def _fini():
        loss_ref[...] = acc_ref[...] * ((1.0 + _COMMIT) / (_N * _D))


def _enc_body(idx_ref, enc_ref, perp_ref, counts_ref):
    i = pl.program_id(0)

    @pl.when(i == 0)
    def _init():
        counts_ref[...] = jnp.zeros_like(counts_ref)

    lanes = jax.lax.broadcasted_iota(jnp.int32, (_TN, _K), 1)
    enc = (lanes == idx_ref[...]).astype(jnp.float32)
    enc_ref[...] = enc
    counts_ref[...] += jnp.sum(enc, axis=0, keepdims=True)

    @pl.when(i == _GRID - 1)
    def _fini():
        avg = counts_ref[...] * (1.0 / _N)
        ent = jnp.sum(avg * jnp.log(avg + 1e-10), keepdims=True)
        perp_ref[...] = jnp.exp(-ent)


def _sc_body(w_hbm, idx_hbm, q_hbm, idx_v, rows_v, sem):
    c = lax.axis_index("c")
    s = lax.axis_index("s")
    wid = c * _NS + s
    base = wid * _BPW

    pltpu.sync_copy(idx_hbm.at[pl.ds(base, _BPW)], idx_v)
    pltpu.async_copy(w_hbm.at[idx_v], rows_v, sem).wait()
    pltpu.sync_copy(rows_v, q_hbm.at[pl.ds(base, _BPW)])


_sc_call = functools.partial(
    pl.kernel,
    mesh=plsc.VectorSubcoreMesh(core_axis_name="c", subcore_axis_name="s"),
    out_type=jax.ShapeDtypeStruct((_N, _DP), jnp.float32),
    scratch_types=[
        pltpu.VMEM((_BPW,), jnp.int32),
        pltpu.VMEM((_BPW, _DP), jnp.float32),
        pltpu.SemaphoreType.DMA,
    ],
)(_sc_body)


@jax.jit
def kernel(inputs, w):
    x = jnp.transpose(inputs, (0, 2, 3, 1))
    input_shape = x.shape
    flat = x.reshape(-1, _D)
    xsq = jnp.sum(flat ** 2, axis=1, keepdims=True)
    wsq = jnp.sum(w ** 2, axis=1).reshape(1, _K)

    dist, idx, loss = pl.pallas_call(
        _dist_body,
        grid=(_GRID,),
        in_specs=[
            pl.BlockSpec((_TN, _D), lambda i: (i, 0)),
            pl.BlockSpec((_TN, 1), lambda i: (i, 0)),
            pl.BlockSpec((_K, _D), lambda i: (0, 0)),
            pl.BlockSpec((1, _K), lambda i: (0, 0)),
        ],
        out_specs=[
            pl.BlockSpec((_TN, _K), lambda i: (i, 0)),
            pl.BlockSpec((_TN, 1), lambda i: (i, 0)),
            pl.BlockSpec((1, 1), lambda i: (0, 0)),
        ],
        out_shape=[
            jax.ShapeDtypeStruct((_N, _K), jnp.float32),
            jax.ShapeDtypeStruct((_N, 1), jnp.int32),
            jax.ShapeDtypeStruct((1, 1), jnp.float32),
        ],
        scratch_shapes=[
            pltpu.VMEM((1, 1), jnp.float32),
        ],
    )(flat, xsq, w, wsq)

    enc, perp = pl.pallas_call(
        _enc_body,
        grid=(_GRID,),
        in_specs=[
            pl.BlockSpec((_TN, 1), lambda i: (i, 0)),
        ],
        out_specs=[
            pl.BlockSpec((_TN, _K), lambda i: (i, 0)),
            pl.BlockSpec((1, 1), lambda i: (0, 0)),
        ],
        out_shape=[
            jax.ShapeDtypeStruct((_N, _K), jnp.float32),
            jax.ShapeDtypeStruct((1, 1), jnp.float32),
        ],
        scratch_shapes=[
            pltpu.VMEM((1, _K), jnp.float32),
        ],
    )(idx)

    w_pad = jnp.pad(w, ((0, 0), (0, _DP - _D)))
    q = _sc_call(w_pad, idx.reshape(_N))

    qst = q[:, 0:_D]
    quantized_out = jnp.transpose(qst.reshape(input_shape), (0, 3, 1, 2))
    return (dist, quantized_out, loss[0, 0], enc, idx, perp[0, 0])


# speedup vs baseline: 1.0630x; 1.0630x over previous
import functools

import jax
import jax.numpy as jnp
from jax import lax
from jax.experimental import pallas as pl
from jax.experimental.pallas import tpu as pltpu
from jax.experimental.pallas import tpu_sc as plsc

_D = 32
_K = 8192
_N = 8192
_TN = 256
_GRID = _N // _TN
_COMMIT = 0.25

_NC = 2
_NS = 16
_NW = _NC * _NS
_BPW = _N // _NW


def _dist_body(flat_ref, xsq_ref, w_ref, wsq_ref,
               dist_ref, idx_ref, enc_ref, perp_ref, loss_ref,
               counts_ref, acc_ref):
    i = pl.program_id(0)

    @pl.when(i == 0)
    def _init():
        counts_ref[...] = jnp.zeros_like(counts_ref)
        acc_ref[...] = jnp.zeros_like(acc_ref)

    x = flat_ref[...]
    w = w_ref[...]
    dots = jax.lax.dot_general(x, w, (((1,), (1,)), ((), ())),
                               preferred_element_type=jnp.float32)
    dist = (xsq_ref[...] - 2.0 * dots) + wsq_ref[...]
    dist_ref[...] = dist

    dmin = jnp.min(dist, axis=1, keepdims=True)
    lanes = jax.lax.broadcasted_iota(jnp.int32, (_TN, _K), 1)
    idx = jnp.min(jnp.where(dist == dmin, lanes, _K), axis=1)
    idx_ref[...] = idx[:, None]
    enc = (lanes == idx[:, None]).astype(jnp.float32)
    enc_ref[...] = enc
    counts_ref[...] += jnp.sum(enc, axis=0, keepdims=True)
    acc_ref[...] += jnp.sum(dmin, keepdims=True).reshape(1, 1)

    @pl.when(i == _GRID - 1)
    def _fini():
        avg = counts_ref[...] * (1.0 / _N)
        ent = jnp.sum(avg * jnp.log(avg + 1e-10), keepdims=True)
        perp_ref[...] = jnp.exp(-ent)
        loss_ref[...] = acc_ref[...] * ((1.0 + _COMMIT) / (_N * _D))


_DP = 128


def _sc_body(w_hbm, idx_hbm, q_hbm, idx_v, rows_v, sem):
    c = lax.axis_index("c")
    s = lax.axis_index("s")
    wid = c * _NS + s
    base = wid * _BPW

    pltpu.sync_copy(idx_hbm.at[pl.ds(base, _BPW)], idx_v)
    pltpu.async_copy(w_hbm.at[idx_v], rows_v, sem).wait()
    pltpu.sync_copy(rows_v, q_hbm.at[pl.ds(base, _BPW)])


_sc_call = functools.partial(
    pl.kernel,
    mesh=plsc.VectorSubcoreMesh(core_axis_name="c", subcore_axis_name="s"),
    out_type=jax.ShapeDtypeStruct((_N, _DP), jnp.float32),
    scratch_types=[
        pltpu.VMEM((_BPW,), jnp.int32),
        pltpu.VMEM((_BPW, _DP), jnp.float32),
        pltpu.SemaphoreType.DMA,
    ],
)(_sc_body)


@jax.jit
def kernel(inputs, w):
    x = jnp.transpose(inputs, (0, 2, 3, 1))
    input_shape = x.shape
    flat = x.reshape(-1, _D)
    xsq = jnp.sum(flat ** 2, axis=1, keepdims=True)
    wsq = jnp.sum(w ** 2, axis=1).reshape(1, _K)

    dist, idx, enc, perp, loss = pl.pallas_call(
        _dist_body,
        grid=(_GRID,),
        in_specs=[
            pl.BlockSpec((_TN, _D), lambda i: (i, 0)),
            pl.BlockSpec((_TN, 1), lambda i: (i, 0)),
            pl.BlockSpec((_K, _D), lambda i: (0, 0)),
            pl.BlockSpec((1, _K), lambda i: (0, 0)),
        ],
        out_specs=[
            pl.BlockSpec((_TN, _K), lambda i: (i, 0)),
            pl.BlockSpec((_TN, 1), lambda i: (i, 0)),
            pl.BlockSpec((_TN, _K), lambda i: (i, 0)),
            pl.BlockSpec((1, 1), lambda i: (0, 0)),
            pl.BlockSpec((1, 1), lambda i: (0, 0)),
        ],
        out_shape=[
            jax.ShapeDtypeStruct((_N, _K), jnp.float32),
            jax.ShapeDtypeStruct((_N, 1), jnp.int32),
            jax.ShapeDtypeStruct((_N, _K), jnp.float32),
            jax.ShapeDtypeStruct((1, 1), jnp.float32),
            jax.ShapeDtypeStruct((1, 1), jnp.float32),
        ],
        scratch_shapes=[
            pltpu.VMEM((1, _K), jnp.float32),
            pltpu.VMEM((1, 1), jnp.float32),
        ],
    )(flat, xsq, w, wsq)

    w_pad = jnp.pad(w, ((0, 0), (0, _DP - _D)))
    q = _sc_call(w_pad, idx.reshape(_N))

    qst = q[:, 0:_D]
    quantized_out = jnp.transpose(qst.reshape(input_shape), (0, 3, 1, 2))
    return (dist, quantized_out, loss[0, 0], enc, idx, perp[0, 0])
